# baseline (speedup 1.0000x reference)
import functools

import jax
import jax.numpy as jnp
from jax import lax
from jax.experimental import pallas as pl
from jax.experimental.pallas import tpu as pltpu
from jax.experimental.pallas import tpu_sc as plsc

T = 8192
D = 2048
E = 64
K = 8
BTM = 2048
NW = 32
CT = T // NW


def _matmul_block(x_ref, w_ref, b_ref, scores_ref):
    scores_ref[...] = (
        jax.lax.dot_general(
            x_ref[...], w_ref[...], (((1,), (1,)), ((), ())),
            preferred_element_type=jnp.float32,
        )
        + b_ref[...]
    )


def _tc_scores(inputs, W, b2):
    return pl.pallas_call(
        _matmul_block,
        grid=(T // BTM,),
        in_specs=[
            pl.BlockSpec((BTM, D), lambda i: (i, 0)),
            pl.BlockSpec((E, D), lambda i: (0, 0)),
            pl.BlockSpec((1, E), lambda i: (0, 0)),
        ],
        out_specs=pl.BlockSpec((BTM, E), lambda i: (i, 0)),
        out_shape=jax.ShapeDtypeStruct((T, E), jnp.float32),
    )(inputs, W, b2)


def _sc_route_kernel(scores_hbm, router_hbm, indices_hbm, sv, rv, iv):
    wid = lax.axis_index("s") * 2 + lax.axis_index("c")
    base = wid * CT
    pltpu.sync_copy(scores_hbm.at[pl.ds(base, CT)], sv)

    iota16 = lax.iota(jnp.int32, 16)
    neg_inf = jnp.full((16,), -jnp.inf, jnp.float32)
    big = jnp.full((16,), 64, jnp.int32)
    perms = [iota16 ^ s for s in (1, 2, 4, 8)]
    _dnums = lax.GatherDimensionNumbers(
        offset_dims=(), collapsed_slice_dims=(0,), start_index_map=(0,))

    def shuffle(x, p):
        return lax.gather(
            x, p[:, None], _dnums, slice_sizes=(1,),
            mode=lax.GatherScatterMode.PROMISE_IN_BOUNDS)

    def lanemax(x):
        for p in perms:
            x = jnp.maximum(x, shuffle(x, p))
        return x

    def lanesum(x):
        for p in perms:
            x = x + shuffle(x, p)
        return x

    def lanemin_i32(x):
        for p in perms:
            x = jnp.minimum(x, shuffle(x, p))
        return x

    one16i = jnp.ones((16,), jnp.int32)
    zero16i = jnp.zeros((16,), jnp.int32)
    zero16f = jnp.zeros((16,), jnp.float32)
    gidx = [iota16 + 16 * j for j in range(4)]

    def body(t, carry):
        v = [sv[t, pl.ds(16 * j, 16)] for j in range(4)]
        a = [one16i for _ in range(4)]
        r0b = None
        for it in range(K):
            mv = [jnp.where(a[j] > zero16i, v[j], neg_inf) for j in range(4)]
            mall = jnp.maximum(jnp.maximum(mv[0], mv[1]),
                               jnp.maximum(mv[2], mv[3]))
            rb = lanemax(mall)
            if it == 0:
                r0b = rb
            cand = [jnp.where(mv[j] == rb, gidx[j], big) for j in range(4)]
            g = lanemin_i32(jnp.minimum(jnp.minimum(cand[0], cand[1]),
                                        jnp.minimum(cand[2], cand[3])))
            for j in range(4):
                a[j] = jnp.where(gidx[j] == g, zero16i, a[j])
        kp = [one16i - a[j] for j in range(4)]
        ex = [jnp.where(kp[j] > zero16i, jnp.exp(v[j] - r0b), zero16f)
              for j in range(4)]
        tot = ex[0] + ex[1] + ex[2] + ex[3]
        sb = lanesum(tot)
        for j in range(4):
            rv[t, pl.ds(16 * j, 16)] = ex[j] / sb
            iv[t, pl.ds(16 * j, 16)] = kp[j].astype(jnp.float32)
        return carry

    lax.fori_loop(0, CT, body, 0)
    pltpu.sync_copy(rv, router_hbm.at[pl.ds(base, CT)])
    pltpu.sync_copy(iv, indices_hbm.at[pl.ds(base, CT)])


def kernel(inputs, W, b):
    b2 = b.reshape(1, E)
    scores = _tc_scores(inputs, W, b2)
    route = functools.partial(
        pl.kernel,
        out_type=[
            jax.ShapeDtypeStruct((T, E), jnp.float32),
            jax.ShapeDtypeStruct((T, E), jnp.float32),
        ],
        mesh=plsc.VectorSubcoreMesh(core_axis_name="c", subcore_axis_name="s"),
        scratch_types=[
            pltpu.VMEM((CT, E), jnp.float32),
            pltpu.VMEM((CT, E), jnp.float32),
            pltpu.VMEM((CT, E), jnp.float32),
        ],
    )(_sc_route_kernel)
    router, indices = route(scores)
    return (router, indices)


# speedup vs baseline: 2.6928x; 2.6928x over previous
import jax
import jax.numpy as jnp
from jax.experimental import pallas as pl
from jax.experimental.pallas import tpu as pltpu

T = 8192
D = 2048
E = 64
K = 8
BT = 1024
DH = D // 2


def _router_block(xa_ref, xb_ref, w_ref, b_ref, router_ref, indices_ref):
    w = w_ref[...]
    dims = (((1,), (1,)), ((), ()))
    scores = (
        jax.lax.dot_general(w[:, :DH], xa_ref[...], dims,
                            preferred_element_type=jnp.float32)
        + jax.lax.dot_general(w[:, DH:], xb_ref[...], dims,
                              preferred_element_type=jnp.float32)
        + b_ref[...].T
    )

    eidx = jax.lax.broadcasted_iota(jnp.int32, scores.shape, 0)
    active = jnp.ones(scores.shape, dtype=jnp.bool_)
    neg_inf = jnp.float32(-jnp.inf)
    rowmax = None
    for it in range(K):
        masked = jnp.where(active, scores, neg_inf)
        m = jnp.max(masked, axis=0, keepdims=True)
        if it == 0:
            rowmax = m
        is_m = active & (scores == m)
        cand = jnp.where(is_m, eidx, E)
        j = jnp.min(cand, axis=0, keepdims=True)
        active = active & (eidx != j)
    keep = jnp.logical_not(active)

    expv = jnp.where(keep, jnp.exp(scores - rowmax), 0.0)
    router_ref[...] = expv / jnp.sum(expv, axis=0, keepdims=True)
    indices_ref[...] = keep.astype(jnp.float32)


def kernel(inputs, W, b):
    b2 = b.reshape(1, E)
    grid = (T // BT,)
    router_t, indices_t = pl.pallas_call(
        _router_block,
        grid=grid,
        in_specs=[
            pl.BlockSpec((BT, DH), lambda i: (i, 0)),
            pl.BlockSpec((BT, DH), lambda i: (i, 1)),
            pl.BlockSpec((E, D), lambda i: (0, 0)),
            pl.BlockSpec((1, E), lambda i: (0, 0)),
        ],
        out_specs=[
            pl.BlockSpec((E, BT), lambda i: (0, i)),
            pl.BlockSpec((E, BT), lambda i: (0, i)),
        ],
        out_shape=[
            jax.ShapeDtypeStruct((E, T), jnp.float32),
            jax.ShapeDtypeStruct((E, T), jnp.float32),
        ],
        compiler_params=pltpu.CompilerParams(
            dimension_semantics=("parallel",),
        ),
    )(inputs, inputs, W, b2)
    return (router_t.T, indices_t.T)
